# Initial kernel scaffold; baseline (speedup 1.0000x reference)
import jax
import jax.numpy as jnp
from jax.experimental import pallas as pl


def kernel(x, edge_index_side, edge_index_upd, W1_side, b1_side, W1_upd, b1_upd, W2_side, b2_side, W2_upd, b2_upd):
    raise NotImplementedError("write your pallas kernel here")


# speedup vs baseline: 4.6946x; 4.6946x over previous
import functools

import jax
import jax.numpy as jnp
from jax import lax
from jax.experimental import pallas as pl
from jax.experimental.pallas import tpu as pltpu
from jax.experimental.pallas import tpu_sc as plsc

N = 10000
F = 128
E = 320000
CHUNK = 128
N_TILES = 16
N_CHUNKS = 2560
E_PAD = N_CHUNKS * CHUNK
CPT = N_CHUNKS // N_TILES
GC = 32
TRASH = N
N_ACC = 10240
N_H = 10240
H_STRIPE = N_H // N_TILES
O_STRIPE = N_ACC // N_TILES

_mesh = plsc.VectorSubcoreMesh(core_axis_name="c", subcore_axis_name="s")


@jax.jit
def _deg(idx4, ones_v):
    @functools.partial(
        pl.kernel,
        mesh=_mesh,
        out_type=jax.ShapeDtypeStruct((4, N_H), jnp.float32),
        scratch_types=[
            pltpu.VMEM((CPT, CHUNK), jnp.int32),
            pltpu.VMEM((CHUNK,), jnp.float32),
            pltpu.VMEM((H_STRIPE,), jnp.float32),
            pltpu.VMEM_SHARED((N_H,), jnp.float32),
            pltpu.VMEM_SHARED((N_H,), jnp.float32),
        ],
    )
    def k(idx_hbm, ones_hbm, deg_hbm, idx_v, ones_vm, zb, hist0, hist1):
        c = lax.axis_index("c")
        s = lax.axis_index("s")
        pltpu.sync_copy(ones_hbm, ones_vm)
        @pl.loop(0, H_STRIPE, step=16)
        def _(i):
            zb[pl.ds(i, 16)] = jnp.zeros((16,), jnp.float32)

        for kk, hist in ((0, hist0), (1, hist1)):
            pltpu.sync_copy(zb, hist.at[pl.ds(s * H_STRIPE, H_STRIPE)])
        plsc.subcore_barrier()
        for kk, hist in ((0, hist0), (1, hist1)):
            pltpu.sync_copy(idx_hbm.at[2 * c + kk].at[pl.ds(s * CPT, CPT)], idx_v)

            @pl.loop(0, CPT)
            def _(j):
                pltpu.sync_copy(ones_vm, hist.at[idx_v.at[j]], add=True)

        plsc.subcore_barrier()
        for kk, hist in ((0, hist0), (1, hist1)):
            pltpu.sync_copy(hist.at[pl.ds(s * H_STRIPE, H_STRIPE)],
                            deg_hbm.at[2 * c + kk].at[pl.ds(s * H_STRIPE, H_STRIPE)])

    return k(idx4, ones_v)


@jax.jit
def _spmv(tables, srcg, dstg):
    @functools.partial(
        pl.kernel,
        mesh=_mesh,
        out_type=jax.ShapeDtypeStruct((2, N_ACC, F), jnp.float32),
        scratch_types=[
            pltpu.VMEM((GC, CHUNK), jnp.int32),
            pltpu.VMEM((GC, CHUNK), jnp.int32),
            pltpu.VMEM((CHUNK, F), jnp.float32),
            pltpu.VMEM((CHUNK, F), jnp.float32),
            pltpu.VMEM((16, F), jnp.float32),
            pltpu.VMEM_SHARED((N_ACC, F), jnp.float32),
            pltpu.SemaphoreType.DMA,
            pltpu.SemaphoreType.DMA,
        ],
    )
    def k(t_hbm, src_hbm, dst_hbm, out_hbm,
          src_v, dst_v, rows_a, rows_b, zb, acc, sem_a, sem_b):
        c = lax.axis_index("c")
        s = lax.axis_index("s")
        table = t_hbm.at[c]
        out = out_hbm.at[c]

        @pl.loop(0, F * 16, step=16)
        def _(i):
            r = i // F
            col = i - r * F
            zb[r, pl.ds(col, 16)] = jnp.zeros((16,), jnp.float32)

        base = s * O_STRIPE

        @pl.loop(0, O_STRIPE, step=16)
        def _(z):
            pltpu.sync_copy(zb, acc.at[pl.ds(base + z, 16)])

        plsc.subcore_barrier()

        @pl.loop(0, CPT // GC)
        def _(g):
            gbase = s * CPT + g * GC
            pltpu.sync_copy(src_hbm.at[c].at[pl.ds(gbase, GC)], src_v)
            pltpu.sync_copy(dst_hbm.at[c].at[pl.ds(gbase, GC)], dst_v)
            pltpu.async_copy(table.at[src_v.at[0]], rows_a, sem_a)
            pltpu.async_copy(table.at[src_v.at[1]], rows_b, sem_b)

            @pl.loop(0, GC, step=2)
            def _(j):
                for b, (rows, sem) in enumerate(((rows_a, sem_a),
                                                 (rows_b, sem_b))):
                    jj = j + b
                    pltpu.make_async_copy(table.at[src_v.at[jj]], rows,
                                          sem).wait()
                    pltpu.sync_copy(rows, acc.at[dst_v.at[jj]], add=True)
                    nxt = jj + 2

                    @pl.when(nxt < GC)
                    def _():
                        pltpu.async_copy(table.at[src_v.at[nxt]], rows, sem)

        plsc.subcore_barrier()
        pltpu.sync_copy(acc.at[pl.ds(s * O_STRIPE, O_STRIPE)],
                        out.at[pl.ds(s * O_STRIPE, O_STRIPE)])

    return k(tables, srcg, dstg)


def _norms(deg_ref):
    nrm = lax.rsqrt(jnp.maximum(deg_ref[...], 1.0))
    return (nrm[:, 0:1], nrm[:, 1:2], nrm[:, 2:3], nrm[:, 3:4])


@jax.jit
def _tc_pre(x, deg_t):
    def body(x_ref, d_ref, o_ref):
        nss, _, nsu, _ = _norms(d_ref)
        xv = x_ref[...]
        o_ref[0] = xv * nss
        o_ref[1] = xv * nsu

    return pl.pallas_call(
        body,
        out_shape=jax.ShapeDtypeStruct((2, N, F), jnp.float32),
    )(x, deg_t)


@jax.jit
def _tc_mid(agg, deg_t, W1s, W1u, b1):
    def body(a_ref, d_ref, ws_ref, wu_ref, b_ref, o_ref):
        nss, nds, nsu, ndu = _norms(d_ref)
        h = jnp.dot(a_ref[0] * nds, ws_ref[...],
                    preferred_element_type=jnp.float32)
        h += jnp.dot(a_ref[1] * ndu, wu_ref[...],
                     preferred_element_type=jnp.float32)
        h = jnp.maximum(h + b_ref[...], 0.0)
        o_ref[0] = h * nss
        o_ref[1] = h * nsu

    return pl.pallas_call(
        body,
        out_shape=jax.ShapeDtypeStruct((2, N, F), jnp.float32),
    )(agg, deg_t, W1s, W1u, b1.reshape(1, F))


@jax.jit
def _tc_out(agg, deg_t, W2s, W2u, b2):
    def body(a_ref, d_ref, ws_ref, wu_ref, b_ref, o_ref):
        _, nds, _, ndu = _norms(d_ref)
        o = jnp.dot(a_ref[0] * nds, ws_ref[...],
                    preferred_element_type=jnp.float32)
        o += jnp.dot(a_ref[1] * ndu, wu_ref[...],
                     preferred_element_type=jnp.float32)
        o_ref[...] = o + b_ref[...]

    return pl.pallas_call(
        body,
        out_shape=jax.ShapeDtypeStruct((N, F), jnp.float32),
    )(agg, deg_t, W2s, W2u, b2.reshape(1, F))


def kernel(x, edge_index_side, edge_index_upd,
           W1_side, b1_side, W1_upd, b1_upd,
           W2_side, b2_side, W2_upd, b2_upd):
    src_s = edge_index_side[0].astype(jnp.int32)
    dst_s = edge_index_side[1].astype(jnp.int32)
    src_u = edge_index_upd[0].astype(jnp.int32)
    dst_u = edge_index_upd[1].astype(jnp.int32)

    def pad(a, fill):
        return jnp.concatenate(
            [a, jnp.full((E_PAD - E,), fill, jnp.int32)]).reshape(N_CHUNKS, CHUNK)

    idx4 = jnp.stack([pad(src_s, TRASH), pad(dst_s, TRASH),
                      pad(src_u, TRASH), pad(dst_u, TRASH)])
    srcg = jnp.stack([pad(src_s, 0), pad(src_u, 0)])
    dstg = jnp.stack([pad(dst_s, TRASH), pad(dst_u, TRASH)])

    degs = _deg(idx4, jnp.ones((CHUNK,), jnp.float32))
    deg_t = degs.T[:N]

    xs = _tc_pre(x, deg_t)
    agg1 = _spmv(xs, srcg, dstg)[:, :N]
    hs = _tc_mid(agg1, deg_t, W1_side, W1_upd, b1_side + b1_upd)
    agg2 = _spmv(hs, srcg, dstg)[:, :N]
    return _tc_out(agg2, deg_t, W2_side, W2_upd, b2_side + b2_upd)
